# baseline (speedup 1.0000x reference)
import functools

import jax
import jax.numpy as jnp
from jax import lax
from jax.experimental import pallas as pl
from jax.experimental.pallas import tpu as pltpu
from jax.experimental.pallas import tpu_sc as plsc

_L = 16


def _make_glove(B, V, D, nc, ns):
    nw = nc * ns
    assert B % nw == 0
    bpw = B // nw
    assert D % _L == 0
    nd = D // _L
    ng = bpw // _L

    mesh = plsc.VectorSubcoreMesh(core_axis_name="c", subcore_axis_name="s")

    @functools.partial(
        pl.kernel,
        out_type=jax.ShapeDtypeStruct((B,), jnp.float32),
        mesh=mesh,
        compiler_params=pltpu.CompilerParams(
            needs_layout_passes=False, use_tc_tiling_on_sc=False),
        scratch_types=[
            pltpu.VMEM((bpw,), jnp.int32),
            pltpu.VMEM((bpw,), jnp.int32),
            pltpu.VMEM((bpw, D), jnp.float32),
            pltpu.VMEM((bpw, D), jnp.float32),
            pltpu.VMEM((bpw,), jnp.float32),
            pltpu.VMEM((bpw,), jnp.float32),
            pltpu.VMEM((_L, _L), jnp.float32),
            pltpu.VMEM((bpw,), jnp.float32),
            pltpu.SemaphoreType.DMA,
        ],
    )
    def glove(left_h, right_h, lemb_h, lbias_h, remb_h, rbias_h, out_h,
              idx_l, idx_r, l_rows, r_rows, bias_l, bias_r, acc_s, out_v, sem):
        wid = lax.axis_index("s") * nc + lax.axis_index("c")
        base = wid * bpw

        pltpu.sync_copy(left_h.at[pl.ds(base, bpw)], idx_l)
        pltpu.sync_copy(right_h.at[pl.ds(base, bpw)], idx_r)

        cps = [
            pltpu.async_copy(lemb_h.at[idx_l], l_rows, sem),
            pltpu.async_copy(remb_h.at[idx_r], r_rows, sem),
            pltpu.async_copy(lbias_h.at[idx_l], bias_l, sem),
            pltpu.async_copy(rbias_h.at[idx_r], bias_r, sem),
        ]
        for cp in cps:
            cp.wait()

        lane = lax.iota(jnp.int32, _L)

        def group(g, carry):
            p0 = g * _L
            for j in range(_L):
                p = p0 + j
                acc = l_rows[p, pl.ds(0, _L)] * r_rows[p, pl.ds(0, _L)]
                for c in range(1, nd):
                    acc = acc + (l_rows[p, pl.ds(c * _L, _L)]
                                 * r_rows[p, pl.ds(c * _L, _L)])
                acc_s[j, pl.ds(0, _L)] = acc
            tot = bias_l[pl.ds(p0, _L)] + bias_r[pl.ds(p0, _L)]
            for d in range(_L):
                tot = tot + plsc.load_gather(
                    acc_s, [lane, jnp.full((_L,), d, jnp.int32)])
            out_v[pl.ds(p0, _L)] = tot
            return carry

        lax.fori_loop(0, ng, group, 0)

        pltpu.sync_copy(out_v, out_h.at[pl.ds(base, bpw)])

    return glove


def kernel(left, right, l_emb, l_bias, r_emb, r_bias):
    B = left.shape[0]
    V, D = l_emb.shape
    info = plsc.get_sparse_core_info()
    fn = _make_glove(B, V, D, info.num_cores, info.num_subcores)
    return fn(
        left.astype(jnp.int32),
        right.astype(jnp.int32),
        l_emb,
        l_bias.reshape(V),
        r_emb,
        r_bias.reshape(V),
    )
